# baseline (speedup 1.0000x reference)
import functools

import jax
import jax.numpy as jnp
from jax import lax
from jax.experimental import pallas as pl
from jax.experimental.pallas import tpu as pltpu
from jax.experimental.pallas import tpu_sc as plsc

N = 10000
E = 320000
D = 128
H = 32
G = 64
NC = 2
NS = 16
NW = NC * NS
EPAD = 327680
CH = 128
TCH = EPAD // CH
C0 = 80
C1 = 80
CMAX = max(C0, C1)
STCH = TCH + CMAX
NPADR = 10240
RPT = NPADR // NS

_mesh = plsc.VectorSubcoreMesh(core_axis_name="c", subcore_axis_name="s")


@functools.partial(
    pl.kernel,
    out_type=jax.ShapeDtypeStruct((NW, NPADR), jnp.float32),
    mesh=_mesh,
    scratch_types=[
        pltpu.VMEM((RPT, 16), jnp.int32),
        pltpu.VMEM((NPADR,), jnp.float32),
    ],
    compiler_params=pltpu.CompilerParams(needs_layout_passes=False),
)
def _hist_kernel(dst_hbm, out_hbm, dstv, hist):
    cid = lax.axis_index("c")
    sid = lax.axis_index("s")
    wid = sid * NC + cid
    pltpu.sync_copy(dst_hbm.at[wid], dstv)

    zf = jnp.zeros((16,), jnp.float32)
    ones = jnp.ones((16,), jnp.float32)

    @pl.loop(0, NPADR // 16)
    def _zero(i):
        hist[pl.ds(i * 16, 16)] = zf

    @pl.loop(0, RPT)
    def _accum(j):
        plsc.addupdate_scatter(hist, [dstv[j]], ones)

    pltpu.sync_copy(hist, out_hbm.at[wid])


@functools.partial(
    pl.kernel,
    out_type=jax.ShapeDtypeStruct((NC, NPADR, H), jnp.float32),
    mesh=_mesh,
    scratch_types=[
        pltpu.VMEM((CMAX, CH), jnp.int32),
        pltpu.VMEM((CMAX, CH), jnp.int32),
        pltpu.VMEM((CH, H), jnp.float32),
        pltpu.VMEM((CH, H), jnp.float32),
        pltpu.VMEM((CH, H), jnp.float32),
        pltpu.VMEM((CH, H), jnp.float32),
        pltpu.VMEM((64, H), jnp.float32),
        pltpu.VMEM((RPT, H), jnp.float32),
        pltpu.VMEM_SHARED((NPADR, H), jnp.float32),
        pltpu.VMEM_SHARED((N, H), jnp.float32),
        pltpu.SemaphoreType.DMA,
        pltpu.SemaphoreType.DMA,
        pltpu.SemaphoreType.DMA,
        pltpu.SemaphoreType.DMA,
        pltpu.SemaphoreType.DMA,
        pltpu.SemaphoreType.DMA,
        pltpu.SemaphoreType.DMA,
        pltpu.SemaphoreType.DMA,
    ],
    compiler_params=pltpu.CompilerParams(
        needs_layout_passes=False, use_tc_tiling_on_sc=False),
)
def _msgpass_kernel(g_hbm, src_hbm, dst_hbm, out_hbm,
                    srcv, dstv, buf0, buf1, buf2, buf3, zblk, rb, acc, gtab,
                    gs0, gs1, gs2, gs3, ss0, ss1, ss2, ss3):
    cid = lax.axis_index("c")
    sid = lax.axis_index("s")
    cnt = jnp.where(cid == 0, C0, C1)
    cbase = jnp.where(cid == 0, sid * C0, NS * C0 + sid * C1)
    pltpu.sync_copy(src_hbm.at[pl.ds(cbase, CMAX)], srcv)
    pltpu.sync_copy(dst_hbm.at[pl.ds(cbase, CMAX)], dstv)

    zf = jnp.zeros((16,), jnp.float32)

    @pl.loop(0, 64)
    def _zb(i):
        zblk[i, pl.ds(0, 16)] = zf
        zblk[i, pl.ds(16, 16)] = zf

    base = sid * RPT
    for t in range(RPT // 64):
        pltpu.sync_copy(zblk, acc.at[pl.ds(base + t * 64, 64)])
    gpt = N // NS
    pltpu.sync_copy(g_hbm.at[pl.ds(sid * gpt, gpt)], rb.at[pl.ds(0, gpt)])
    pltpu.sync_copy(rb.at[pl.ds(0, gpt)], gtab.at[pl.ds(sid * gpt, gpt)])
    plsc.subcore_barrier()

    bufs = [buf0, buf1, buf2, buf3]
    gsems = [gs0, gs1, gs2, gs3]
    ssems = [ss0, ss1, ss2, ss3]
    for b in range(2):
        pltpu.async_copy(gtab.at[srcv.at[b]], bufs[b], gsems[b])

    @pl.loop(0, cnt, step=4)
    def _chunks(j0):
        for b in range(4):
            j = j0 + b
            bn = (b + 2) % 4

            @pl.when(j - 2 >= 0)
            def _wait_scatter():
                pltpu.make_async_copy(
                    bufs[bn], acc.at[dstv.at[j - 2]], ssems[bn]).wait()

            @pl.when(j + 2 < cnt)
            def _issue_gather():
                pltpu.async_copy(gtab.at[srcv.at[j + 2]], bufs[bn], gsems[bn])

            pltpu.make_async_copy(gtab.at[srcv.at[j]], bufs[b], gsems[b]).wait()
            pltpu.async_copy(bufs[b], acc.at[dstv.at[j]], ssems[b], add=True)

    for b in (2, 3):
        j = cnt - 4 + b
        pltpu.make_async_copy(bufs[b], acc.at[dstv.at[j]], ssems[b]).wait()

    plsc.subcore_barrier()
    pltpu.sync_copy(acc.at[pl.ds(base, RPT)], rb)
    pltpu.sync_copy(rb, out_hbm.at[cid, pl.ds(base, RPT)])


def _stage_deg_body(hist_ref, dinv_ref):
    deg = jnp.sum(hist_ref[...], axis=0) + 1.0
    dinv_ref[...] = lax.rsqrt(deg)


def _stage_mm_body(x_ref, w1_ref, h_ref):
    h_ref[...] = jnp.dot(x_ref[...], w1_ref[...],
                         preferred_element_type=jnp.float32)


def _stage_scale_body(h_ref, dinv_ref, g_ref):
    g_ref[...] = h_ref[...] * dinv_ref[...][:N]


def _stage4_body(p_ref, g_ref, dinv_ref, b1_ref, w2_ref, g2_ref):
    dinv = dinv_ref[...][:N]
    agg = p_ref[0, :N, :] + p_ref[1, :N, :] + g_ref[...]
    out1 = jnp.maximum(agg * dinv + b1_ref[...], 0.0)
    h2 = jnp.dot(out1, w2_ref[...], preferred_element_type=jnp.float32)
    g2_ref[...] = h2 * dinv


def _stage6_body(p_ref, g_ref, dinv_ref, b2_ref, batch_ref,
                 hw1_ref, hb1_ref, hw2_ref, hb2_ref, o_ref):
    dinv = dinv_ref[...][:N]
    agg = p_ref[0, :N, :] + p_ref[1, :N, :] + g_ref[...]
    h2 = jnp.maximum(agg * dinv + b2_ref[...], 0.0)
    gid = lax.broadcasted_iota(jnp.int32, (G, N), 0)
    onehot = (gid == batch_ref[...]).astype(jnp.float32)
    seg = jnp.dot(onehot, h2, preferred_element_type=jnp.float32)
    counts = jnp.sum(onehot, axis=1, keepdims=True)
    pooled = seg / jnp.maximum(counts, 1.0)
    z = jnp.maximum(
        jnp.dot(pooled, hw1_ref[...], preferred_element_type=jnp.float32)
        + hb1_ref[...], 0.0)
    o = jnp.dot(z, hw2_ref[...], preferred_element_type=jnp.float32) + hb2_ref[...]
    o_ref[...] = jax.nn.sigmoid(o)


_stage_deg = pl.pallas_call(
    _stage_deg_body,
    out_shape=jax.ShapeDtypeStruct((NPADR // 128, 128), jnp.float32),
)

_stage_mm = pl.pallas_call(
    _stage_mm_body,
    out_shape=jax.ShapeDtypeStruct((N, H), jnp.float32),
)

_stage_scale = pl.pallas_call(
    _stage_scale_body,
    out_shape=jax.ShapeDtypeStruct((N, H), jnp.float32),
)

_stage4 = pl.pallas_call(
    _stage4_body,
    out_shape=jax.ShapeDtypeStruct((N, H), jnp.float32),
)

_stage6 = pl.pallas_call(
    _stage6_body,
    out_shape=jax.ShapeDtypeStruct((G, 1), jnp.float32),
)


def kernel(x, edge_index, batch, W1, b1, W2, b2, Hw1, Hb1, Hw2, Hb2):
    src = edge_index[0].astype(jnp.int32)
    dst = edge_index[1].astype(jnp.int32)
    src_p = jnp.concatenate([src, jnp.zeros((STCH * CH - E,), jnp.int32)])
    dst_p = jnp.concatenate([dst, jnp.full((STCH * CH - E,), N, jnp.int32)])
    src3 = src_p.reshape(STCH, CH)
    dst3 = dst_p.reshape(STCH, CH)
    dsth = dst_p[:EPAD].reshape(NW, RPT, 16)

    h1 = _stage_mm(x, W1)
    hist = _hist_kernel(dsth)
    dinv = _stage_deg(hist.reshape(NW, NPADR // 128, 128))
    dinv = dinv.reshape(NPADR, 1)
    g1 = _stage_scale(h1, dinv)
    p1 = _msgpass_kernel(g1, src3, dst3)
    g2 = _stage4(p1, g1, dinv, b1.reshape(1, H), W2)
    p2 = _msgpass_kernel(g2, src3, dst3)
    out = _stage6(p2, g2, dinv, b2.reshape(1, H),
                  batch.astype(jnp.int32).reshape(1, N),
                  Hw1, Hb1.reshape(1, H), Hw2, Hb2.reshape(1, 1))
    return out.reshape(G)


# speedup vs baseline: 1.0342x; 1.0342x over previous
import functools

import jax
import jax.numpy as jnp
from jax import lax
from jax.experimental import pallas as pl
from jax.experimental.pallas import tpu as pltpu
from jax.experimental.pallas import tpu_sc as plsc

N = 10000
E = 320000
D = 128
H = 32
G = 64
NC = 2
NS = 16
NW = NC * NS
EPAD = 327680
CH = 128
TCH = EPAD // CH
C0 = 80
C1 = 80
CMAX = max(C0, C1)
STCH = TCH + CMAX
NPADR = 10240
RPT = NPADR // NS

_mesh = plsc.VectorSubcoreMesh(core_axis_name="c", subcore_axis_name="s")


@functools.partial(
    pl.kernel,
    out_type=jax.ShapeDtypeStruct((NW, NPADR), jnp.float32),
    mesh=_mesh,
    scratch_types=[
        pltpu.VMEM((RPT, 16), jnp.int32),
        pltpu.VMEM((NPADR,), jnp.float32),
    ],
    compiler_params=pltpu.CompilerParams(needs_layout_passes=False),
)
def _hist_kernel(dst_hbm, out_hbm, dstv, hist):
    cid = lax.axis_index("c")
    sid = lax.axis_index("s")
    wid = sid * NC + cid
    pltpu.sync_copy(dst_hbm.at[wid], dstv)

    zf = jnp.zeros((16,), jnp.float32)
    ones = jnp.ones((16,), jnp.float32)

    @pl.loop(0, NPADR // 16)
    def _zero(i):
        hist[pl.ds(i * 16, 16)] = zf

    @pl.loop(0, RPT)
    def _accum(j):
        plsc.addupdate_scatter(hist, [dstv[j]], ones)

    pltpu.sync_copy(hist, out_hbm.at[wid])


@functools.partial(
    pl.kernel,
    out_type=jax.ShapeDtypeStruct((NC, NPADR, H), jnp.float32),
    mesh=_mesh,
    scratch_types=[
        pltpu.VMEM((CMAX, CH), jnp.int32),
        pltpu.VMEM((CMAX, CH), jnp.int32),
        pltpu.VMEM((CH, H), jnp.float32),
        pltpu.VMEM((CH, H), jnp.float32),
        pltpu.VMEM((CH, H), jnp.float32),
        pltpu.VMEM((CH, H), jnp.float32),
        pltpu.VMEM((64, H), jnp.float32),
        pltpu.VMEM((RPT, H), jnp.float32),
        pltpu.VMEM_SHARED((NPADR, H), jnp.float32),
        pltpu.VMEM_SHARED((N, H), jnp.float32),
        pltpu.SemaphoreType.DMA,
        pltpu.SemaphoreType.DMA,
        pltpu.SemaphoreType.DMA,
        pltpu.SemaphoreType.DMA,
        pltpu.SemaphoreType.DMA,
        pltpu.SemaphoreType.DMA,
        pltpu.SemaphoreType.DMA,
        pltpu.SemaphoreType.DMA,
    ],
    compiler_params=pltpu.CompilerParams(
        needs_layout_passes=False, use_tc_tiling_on_sc=False),
)
def _msgpass_kernel(g_hbm, src_hbm, dst_hbm, out_hbm,
                    srcv, dstv, buf0, buf1, buf2, buf3, zblk, rb, acc, gtab,
                    gs0, gs1, gs2, gs3, ss0, ss1, ss2, ss3):
    cid = lax.axis_index("c")
    sid = lax.axis_index("s")
    cnt = jnp.where(cid == 0, C0, C1)
    cbase = jnp.where(cid == 0, sid * C0, NS * C0 + sid * C1)
    bufs = [buf0, buf1, buf2, buf3]
    gsems = [gs0, gs1, gs2, gs3]
    ssems = [ss0, ss1, ss2, ss3]

    c_src = pltpu.async_copy(src_hbm.at[pl.ds(cbase, CMAX)], srcv, gs0)
    c_dst = pltpu.async_copy(dst_hbm.at[pl.ds(cbase, CMAX)], dstv, gs1)
    gpt = N // NS
    c_g = pltpu.async_copy(g_hbm.at[pl.ds(sid * gpt, gpt)],
                           rb.at[pl.ds(0, gpt)], gs2)

    zf = jnp.zeros((16,), jnp.float32)

    @pl.loop(0, 64)
    def _zb(i):
        zblk[i, pl.ds(0, 16)] = zf
        zblk[i, pl.ds(16, 16)] = zf

    base = sid * RPT
    nzb = RPT // 64
    for t in range(nzb):
        if t >= 4:
            pltpu.make_async_copy(
                zblk, acc.at[pl.ds(base + (t - 4) * 64, 64)],
                ssems[(t - 4) % 4]).wait()
        pltpu.async_copy(zblk, acc.at[pl.ds(base + t * 64, 64)], ssems[t % 4])
    c_g.wait()
    c_g2 = pltpu.async_copy(rb.at[pl.ds(0, gpt)],
                            gtab.at[pl.ds(sid * gpt, gpt)], gs3)
    c_src.wait()
    c_dst.wait()
    for t in range(nzb - 4, nzb):
        pltpu.make_async_copy(zblk, acc.at[pl.ds(base + t * 64, 64)],
                              ssems[t % 4]).wait()
    c_g2.wait()
    plsc.subcore_barrier()
    for b in range(2):
        pltpu.async_copy(gtab.at[srcv.at[b]], bufs[b], gsems[b])

    @pl.loop(0, cnt, step=4)
    def _chunks(j0):
        for b in range(4):
            j = j0 + b
            bn = (b + 2) % 4

            @pl.when(j - 2 >= 0)
            def _wait_scatter():
                pltpu.make_async_copy(
                    bufs[bn], acc.at[dstv.at[j - 2]], ssems[bn]).wait()

            @pl.when(j + 2 < cnt)
            def _issue_gather():
                pltpu.async_copy(gtab.at[srcv.at[j + 2]], bufs[bn], gsems[bn])

            pltpu.make_async_copy(gtab.at[srcv.at[j]], bufs[b], gsems[b]).wait()
            pltpu.async_copy(bufs[b], acc.at[dstv.at[j]], ssems[b], add=True)

    for b in (2, 3):
        j = cnt - 4 + b
        pltpu.make_async_copy(bufs[b], acc.at[dstv.at[j]], ssems[b]).wait()

    plsc.subcore_barrier()
    pltpu.sync_copy(acc.at[pl.ds(base, RPT)], rb)
    pltpu.sync_copy(rb, out_hbm.at[cid, pl.ds(base, RPT)])


def _stage_deg_body(hist_ref, dinv_ref):
    deg = jnp.sum(hist_ref[...], axis=0) + 1.0
    dinv_ref[...] = lax.rsqrt(deg)


def _stage2_body(x_ref, w1_ref, dinv_ref, g_ref):
    h = jnp.dot(x_ref[...], w1_ref[...], preferred_element_type=jnp.float32)
    g_ref[...] = h * dinv_ref[...][:N]


def _stage4_body(p_ref, g_ref, dinv_ref, b1_ref, w2_ref, g2_ref):
    dinv = dinv_ref[...][:N]
    agg = p_ref[0, :N, :] + p_ref[1, :N, :] + g_ref[...]
    out1 = jnp.maximum(agg * dinv + b1_ref[...], 0.0)
    h2 = jnp.dot(out1, w2_ref[...], preferred_element_type=jnp.float32)
    g2_ref[...] = h2 * dinv


def _stage6_body(p_ref, g_ref, dinv_ref, b2_ref, batch_ref,
                 hw1_ref, hb1_ref, hw2_ref, hb2_ref, o_ref):
    dinv = dinv_ref[...][:N]
    agg = p_ref[0, :N, :] + p_ref[1, :N, :] + g_ref[...]
    h2 = jnp.maximum(agg * dinv + b2_ref[...], 0.0)
    gid = lax.broadcasted_iota(jnp.int32, (G, N), 0)
    onehot = (gid == batch_ref[...]).astype(jnp.float32)
    seg = jnp.dot(onehot, h2, preferred_element_type=jnp.float32)
    counts = jnp.sum(onehot, axis=1, keepdims=True)
    pooled = seg / jnp.maximum(counts, 1.0)
    z = jnp.maximum(
        jnp.dot(pooled, hw1_ref[...], preferred_element_type=jnp.float32)
        + hb1_ref[...], 0.0)
    o = jnp.dot(z, hw2_ref[...], preferred_element_type=jnp.float32) + hb2_ref[...]
    o_ref[...] = jax.nn.sigmoid(o)


_stage_deg = pl.pallas_call(
    _stage_deg_body,
    out_shape=jax.ShapeDtypeStruct((NPADR // 128, 128), jnp.float32),
)

_stage2 = pl.pallas_call(
    _stage2_body,
    out_shape=jax.ShapeDtypeStruct((N, H), jnp.float32),
)

_stage4 = pl.pallas_call(
    _stage4_body,
    out_shape=jax.ShapeDtypeStruct((N, H), jnp.float32),
)

_stage6 = pl.pallas_call(
    _stage6_body,
    out_shape=jax.ShapeDtypeStruct((G, 1), jnp.float32),
)


def kernel(x, edge_index, batch, W1, b1, W2, b2, Hw1, Hb1, Hw2, Hb2):
    src = edge_index[0].astype(jnp.int32)
    dst = edge_index[1].astype(jnp.int32)
    src_p = jnp.concatenate([src, jnp.zeros((STCH * CH - E,), jnp.int32)])
    dst_p = jnp.concatenate([dst, jnp.full((STCH * CH - E,), N, jnp.int32)])
    src3 = src_p.reshape(STCH, CH)
    dst3 = dst_p.reshape(STCH, CH)
    dsth = dst_p[:EPAD].reshape(NW, RPT, 16)

    hist = _hist_kernel(dsth)
    dinv = _stage_deg(hist.reshape(NW, NPADR // 128, 128))
    dinv = dinv.reshape(NPADR, 1)
    g1 = _stage2(x, W1, dinv)
    p1 = _msgpass_kernel(g1, src3, dst3)
    g2 = _stage4(p1, g1, dinv, b1.reshape(1, H), W2)
    p2 = _msgpass_kernel(g2, src3, dst3)
    out = _stage6(p2, g2, dinv, b2.reshape(1, H),
                  batch.astype(jnp.int32).reshape(1, N),
                  Hw1, Hb1.reshape(1, H), Hw2, Hb2.reshape(1, 1))
    return out.reshape(G)


# speedup vs baseline: 1.0477x; 1.0130x over previous
import functools

import jax
import jax.numpy as jnp
from jax import lax
from jax.experimental import pallas as pl
from jax.experimental.pallas import tpu as pltpu
from jax.experimental.pallas import tpu_sc as plsc

N = 10000
E = 320000
D = 128
H = 32
G = 64
NC = 2
NS = 16
NW = NC * NS
EPAD = 327680
CH = 128
TCH = EPAD // CH
C0 = 80
C1 = 80
CMAX = max(C0, C1)
STCH = TCH + CMAX
NPADR = 10240
RPT = NPADR // NS

_mesh = plsc.VectorSubcoreMesh(core_axis_name="c", subcore_axis_name="s")


@functools.partial(
    pl.kernel,
    out_type=jax.ShapeDtypeStruct((NW, NPADR), jnp.float32),
    mesh=_mesh,
    scratch_types=[
        pltpu.VMEM((RPT, 16), jnp.int32),
        pltpu.VMEM((NPADR,), jnp.float32),
        pltpu.SemaphoreType.DMA,
    ],
    compiler_params=pltpu.CompilerParams(needs_layout_passes=False),
)
def _hist_kernel(dst_hbm, out_hbm, dstv, hist, sem):
    cid = lax.axis_index("c")
    sid = lax.axis_index("s")
    wid = sid * NC + cid
    c_idx = pltpu.async_copy(dst_hbm.at[wid], dstv, sem)

    zf = jnp.zeros((16,), jnp.float32)
    ones = jnp.ones((16,), jnp.float32)

    @pl.loop(0, NPADR // 16)
    def _zero(i):
        hist[pl.ds(i * 16, 16)] = zf

    c_idx.wait()

    @pl.loop(0, RPT)
    def _accum(j):
        plsc.addupdate_scatter(hist, [dstv[j]], ones)

    pltpu.sync_copy(hist, out_hbm.at[wid])


@functools.partial(
    pl.kernel,
    out_type=jax.ShapeDtypeStruct((NC, NPADR, H), jnp.float32),
    mesh=_mesh,
    scratch_types=[
        pltpu.VMEM((CMAX, CH), jnp.int32),
        pltpu.VMEM((CMAX, CH), jnp.int32),
        pltpu.VMEM((CH, H), jnp.float32),
        pltpu.VMEM((CH, H), jnp.float32),
        pltpu.VMEM((CH, H), jnp.float32),
        pltpu.VMEM((CH, H), jnp.float32),
        pltpu.VMEM((64, H), jnp.float32),
        pltpu.VMEM((RPT, H), jnp.float32),
        pltpu.VMEM_SHARED((NPADR, H), jnp.float32),
        pltpu.VMEM_SHARED((N, H), jnp.float32),
        pltpu.SemaphoreType.DMA,
        pltpu.SemaphoreType.DMA,
        pltpu.SemaphoreType.DMA,
        pltpu.SemaphoreType.DMA,
        pltpu.SemaphoreType.DMA,
        pltpu.SemaphoreType.DMA,
        pltpu.SemaphoreType.DMA,
        pltpu.SemaphoreType.DMA,
    ],
    compiler_params=pltpu.CompilerParams(
        needs_layout_passes=False, use_tc_tiling_on_sc=False),
)
def _msgpass_kernel(g_hbm, src_hbm, dst_hbm, out_hbm,
                    srcv, dstv, buf0, buf1, buf2, buf3, zblk, rb, acc, gtab,
                    gs0, gs1, gs2, gs3, ss0, ss1, ss2, ss3):
    cid = lax.axis_index("c")
    sid = lax.axis_index("s")
    cnt = jnp.where(cid == 0, C0, C1)
    cbase = jnp.where(cid == 0, sid * C0, NS * C0 + sid * C1)
    bufs = [buf0, buf1, buf2, buf3]
    gsems = [gs0, gs1, gs2, gs3]
    ssems = [ss0, ss1, ss2, ss3]

    c_src = pltpu.async_copy(src_hbm.at[pl.ds(cbase, CMAX)], srcv, gs0)
    c_dst = pltpu.async_copy(dst_hbm.at[pl.ds(cbase, CMAX)], dstv, gs1)
    gpt = N // NS
    c_g = pltpu.async_copy(g_hbm.at[pl.ds(sid * gpt, gpt)],
                           rb.at[pl.ds(0, gpt)], gs2)

    zf = jnp.zeros((16,), jnp.float32)

    @pl.loop(0, 64)
    def _zb(i):
        zblk[i, pl.ds(0, 16)] = zf
        zblk[i, pl.ds(16, 16)] = zf

    base = sid * RPT
    nzb = RPT // 64
    for t in range(nzb):
        if t >= 4:
            pltpu.make_async_copy(
                zblk, acc.at[pl.ds(base + (t - 4) * 64, 64)],
                ssems[(t - 4) % 4]).wait()
        pltpu.async_copy(zblk, acc.at[pl.ds(base + t * 64, 64)], ssems[t % 4])
    c_g.wait()
    c_g2 = pltpu.async_copy(rb.at[pl.ds(0, gpt)],
                            gtab.at[pl.ds(sid * gpt, gpt)], gs3)
    c_src.wait()
    c_dst.wait()
    for t in range(nzb - 4, nzb):
        pltpu.make_async_copy(zblk, acc.at[pl.ds(base + t * 64, 64)],
                              ssems[t % 4]).wait()
    c_g2.wait()
    plsc.subcore_barrier()
    for b in range(2):
        pltpu.async_copy(gtab.at[srcv.at[b]], bufs[b], gsems[b])

    @pl.loop(0, cnt, step=4)
    def _chunks(j0):
        for b in range(4):
            j = j0 + b
            bn = (b + 2) % 4

            @pl.when(j - 2 >= 0)
            def _wait_scatter():
                pltpu.make_async_copy(
                    bufs[bn], acc.at[dstv.at[j - 2]], ssems[bn]).wait()

            @pl.when(j + 2 < cnt)
            def _issue_gather():
                pltpu.async_copy(gtab.at[srcv.at[j + 2]], bufs[bn], gsems[bn])

            pltpu.make_async_copy(gtab.at[srcv.at[j]], bufs[b], gsems[b]).wait()
            pltpu.async_copy(bufs[b], acc.at[dstv.at[j]], ssems[b], add=True)

    for b in (2, 3):
        j = cnt - 4 + b
        pltpu.make_async_copy(bufs[b], acc.at[dstv.at[j]], ssems[b]).wait()

    plsc.subcore_barrier()
    pltpu.sync_copy(acc.at[pl.ds(base, RPT)], rb)
    pltpu.sync_copy(rb, out_hbm.at[cid, pl.ds(base, RPT)])


def _stage_deg_body(hist_ref, dinv_ref):
    deg = jnp.sum(hist_ref[...], axis=0) + 1.0
    dinv_ref[...] = lax.rsqrt(deg)


def _stage2_body(x_ref, w1_ref, dinv_ref, g_ref):
    h = jnp.dot(x_ref[...], w1_ref[...], preferred_element_type=jnp.float32)
    g_ref[...] = h * dinv_ref[...][:N]


def _stage4_body(p_ref, g_ref, dinv_ref, b1_ref, w2_ref, g2_ref):
    dinv = dinv_ref[...][:N]
    agg = p_ref[0, :N, :] + p_ref[1, :N, :] + g_ref[...]
    out1 = jnp.maximum(agg * dinv + b1_ref[...], 0.0)
    h2 = jnp.dot(out1, w2_ref[...], preferred_element_type=jnp.float32)
    g2_ref[...] = h2 * dinv


def _stage6_body(p_ref, g_ref, dinv_ref, b2_ref, batch_ref,
                 hw1_ref, hb1_ref, hw2_ref, hb2_ref, o_ref):
    dinv = dinv_ref[...][:N]
    agg = p_ref[0, :N, :] + p_ref[1, :N, :] + g_ref[...]
    h2 = jnp.maximum(agg * dinv + b2_ref[...], 0.0)
    gid = lax.broadcasted_iota(jnp.int32, (G, N), 0)
    onehot = (gid == batch_ref[...]).astype(jnp.float32)
    seg = jnp.dot(onehot, h2, preferred_element_type=jnp.float32)
    counts = jnp.sum(onehot, axis=1, keepdims=True)
    pooled = seg / jnp.maximum(counts, 1.0)
    z = jnp.maximum(
        jnp.dot(pooled, hw1_ref[...], preferred_element_type=jnp.float32)
        + hb1_ref[...], 0.0)
    o = jnp.dot(z, hw2_ref[...], preferred_element_type=jnp.float32) + hb2_ref[...]
    o_ref[...] = jax.nn.sigmoid(o)


_stage_deg = pl.pallas_call(
    _stage_deg_body,
    out_shape=jax.ShapeDtypeStruct((NPADR // 128, 128), jnp.float32),
)

_stage2 = pl.pallas_call(
    _stage2_body,
    out_shape=jax.ShapeDtypeStruct((N, H), jnp.float32),
)

_stage4 = pl.pallas_call(
    _stage4_body,
    out_shape=jax.ShapeDtypeStruct((N, H), jnp.float32),
)

_stage6 = pl.pallas_call(
    _stage6_body,
    out_shape=jax.ShapeDtypeStruct((G, 1), jnp.float32),
)


def kernel(x, edge_index, batch, W1, b1, W2, b2, Hw1, Hb1, Hw2, Hb2):
    src = edge_index[0].astype(jnp.int32)
    dst = edge_index[1].astype(jnp.int32)
    src_p = jnp.concatenate([src, jnp.zeros((STCH * CH - E,), jnp.int32)])
    dst_p = jnp.concatenate([dst, jnp.full((STCH * CH - E,), N, jnp.int32)])
    src3 = src_p.reshape(STCH, CH)
    dst3 = dst_p.reshape(STCH, CH)
    dsth = dst_p[:EPAD].reshape(NW, RPT, 16)

    hist = _hist_kernel(dsth)
    dinv = _stage_deg(hist.reshape(NW, NPADR // 128, 128))
    dinv = dinv.reshape(NPADR, 1)
    g1 = _stage2(x, W1, dinv)
    p1 = _msgpass_kernel(g1, src3, dst3)
    g2 = _stage4(p1, g1, dinv, b1.reshape(1, H), W2)
    p2 = _msgpass_kernel(g2, src3, dst3)
    out = _stage6(p2, g2, dinv, b2.reshape(1, H),
                  batch.astype(jnp.int32).reshape(1, N),
                  Hw1, Hb1.reshape(1, H), Hw2, Hb2.reshape(1, 1))
    return out.reshape(G)
